# Initial kernel scaffold; baseline (speedup 1.0000x reference)
import jax
import jax.numpy as jnp
from jax.experimental import pallas as pl


def kernel(x, edge_index, edge_attr, W1, b1, W2, b2):
    raise NotImplementedError("write your pallas kernel here")


# speedup vs baseline: 2.1014x; 2.1014x over previous
import functools

import jax
import jax.numpy as jnp
from jax import lax
from jax.experimental import pallas as pl
from jax.experimental.pallas import tpu as pltpu

_N = 10000
_DIN = 128
_DH = 256
_DOUT = 128
_M = 100
_RB = 1000
_NRB = _N // _RB
_HB = 10

_F32 = jnp.float32


def _dinv_from_degT(degT_ref):
    deg = degT_ref[:, 0:1] + degT_ref[:, 1:2]
    safe = jnp.where(deg > 0.0, deg, 1.0)
    return jnp.where(deg > 0.0, lax.rsqrt(safe), 0.0)


def _g1_body(x_ref, w_ref, degT_ref, out_ref):
    dinv = _dinv_from_degT(degT_ref)
    h = jnp.dot(x_ref[...], w_ref[...], preferred_element_type=_F32)
    out_ref[...] = h * dinv


def _mid_body(s1a_ref, s1b_ref, w2a_ref, w2b_ref, b1r_ref, degT_ref, out_ref):
    dinv = _dinv_from_degT(degT_ref)
    z1a = jnp.maximum(s1a_ref[0] * dinv + b1r_ref[0:1, :], 0.0)
    z1b = jnp.maximum(s1b_ref[0] * dinv + b1r_ref[1:2, :], 0.0)
    h2 = (jnp.dot(z1a, w2a_ref[...], preferred_element_type=_F32)
          + jnp.dot(z1b, w2b_ref[...], preferred_element_type=_F32))
    out_ref[...] = h2 * dinv


def _z2_body(s2a_ref, s2b_ref, b2_ref, degT_ref, out_ref):
    dinv = _dinv_from_degT(degT_ref)
    out_ref[...] = (s2a_ref[0] + s2b_ref[0]) * dinv + b2_ref[...]


def _head_body(lat_ref, out_ref):
    for t in range(_HB):
        a = lat_ref[t]
        g = lax.dot_general(a, a, (((1,), (1,)), ((), ())),
                            preferred_element_type=_F32)
        out_ref[t] = 1.0 / (1.0 + jnp.exp(-g))


def _g1cat(x, W1, degT):
    return pl.pallas_call(
        _g1_body,
        grid=(_NRB, 2),
        in_specs=[
            pl.BlockSpec((_RB, _DIN), lambda i, j: (i, 0)),
            pl.BlockSpec((_DIN, 128), lambda i, j: (0, j)),
            pl.BlockSpec((_RB, 2), lambda i, j: (i, 0)),
        ],
        out_specs=pl.BlockSpec((_RB, 128), lambda i, j: (j * _NRB + i, 0)),
        out_shape=jax.ShapeDtypeStruct((2 * _N, 128), _F32),
    )(x, W1, degT)


def _g2(s1, W2, b1r, degT):
    return pl.pallas_call(
        _mid_body,
        grid=(_NRB,),
        in_specs=[
            pl.BlockSpec((1, _RB, 128), lambda i: (0, i, 0)),
            pl.BlockSpec((1, _RB, 128), lambda i: (1, i, 0)),
            pl.BlockSpec((128, 128), lambda i: (0, 0)),
            pl.BlockSpec((128, 128), lambda i: (1, 0)),
            pl.BlockSpec((2, 128), lambda i: (0, 0)),
            pl.BlockSpec((_RB, 2), lambda i: (i, 0)),
        ],
        out_specs=pl.BlockSpec((_RB, 128), lambda i: (i, 0)),
        out_shape=jax.ShapeDtypeStruct((_N, 128), _F32),
    )(s1, s1, W2, W2, b1r, degT)


def _lat(s2, b2r, degT):
    return pl.pallas_call(
        _z2_body,
        grid=(_NRB,),
        in_specs=[
            pl.BlockSpec((1, _RB, 128), lambda i: (0, i, 0)),
            pl.BlockSpec((1, _RB, 128), lambda i: (1, i, 0)),
            pl.BlockSpec((1, 128), lambda i: (0, 0)),
            pl.BlockSpec((_RB, 2), lambda i: (i, 0)),
        ],
        out_specs=pl.BlockSpec((_RB, 128), lambda i: (i, 0)),
        out_shape=jax.ShapeDtypeStruct((_N, 128), _F32),
    )(s2, s2, b2r, degT)


def _head(latB):
    return pl.pallas_call(
        _head_body,
        grid=(_M // _HB,),
        in_specs=[pl.BlockSpec((_HB, _M, _DOUT), lambda i: (i, 0, 0))],
        out_specs=pl.BlockSpec((_HB, _M, _M), lambda i: (i, 0, 0)),
        out_shape=jax.ShapeDtypeStruct((_M, _M, _M), _F32),
    )(latB)


def kernel(x, edge_index, edge_attr, W1, b1, W2, b2):
    src = edge_index[0]
    dst = edge_index[1]
    ew = edge_attr

    deg = jnp.zeros((_N,), _F32).at[dst].add(ew)
    degT = jnp.stack([deg, jnp.zeros_like(deg)], axis=1)

    g1cat = _g1cat(x, W1, degT)
    g1full = jnp.concatenate([g1cat[:_N], g1cat[_N:]], axis=1)
    s1full = jnp.zeros((_N, _DH), _F32).at[dst].add(ew[:, None] * g1full[src])
    s1 = s1full.reshape(_N, 2, 128).transpose(1, 0, 2)

    g2 = _g2(s1, W2, b1.reshape(2, 128), degT)
    s2full = jnp.zeros((_N, _DOUT), _F32).at[dst].add(ew[:, None] * g2[src])
    s2 = jnp.stack([s2full, jnp.zeros_like(s2full)])

    lat = _lat(s2, b2.reshape(1, 128), degT)
    return _head(lat.reshape(_M, _M, _DOUT))


# speedup vs baseline: 5.5300x; 2.6316x over previous
import functools

import jax
import jax.numpy as jnp
from jax import lax
from jax.experimental import pallas as pl
from jax.experimental.pallas import tpu as pltpu
from jax.experimental.pallas import tpu_sc as plsc

_N = 10000
_DIN = 128
_DH = 256
_DOUT = 128
_M = 100
_RB = 1000
_NRB = _N // _RB
_HB = 10

_F32 = jnp.float32

_E = 320000
_EP = 327680
_EROWS = _EP // 128
_RT_COLS = _EROWS // 16
_RT_EDGE = _EROWS // 32
_NP = 10240
_NPT = _NP // 16
_DEG_PAD = _NP

_SC_MESH = plsc.VectorSubcoreMesh(core_axis_name="c", subcore_axis_name="s")


def _sc_deg_body(dst_hbm, ew_hbm, out_hbm, dst_v, ew_v, zbuf, deg_sh):
    c = lax.axis_index("c")
    s = lax.axis_index("s")

    @pl.loop(0, 40)
    def _zero(i):
        zbuf[pl.ds(i * 16, 16)] = jnp.zeros((16,), _F32)

    pltpu.sync_copy(zbuf, deg_sh.at[pl.ds(s * 640, 640)])
    plsc.subcore_barrier()

    row0 = c * (_EROWS // 2) + s * _RT_EDGE
    pltpu.sync_copy(dst_hbm.at[pl.ds(row0, _RT_EDGE)], dst_v)
    pltpu.sync_copy(ew_hbm.at[pl.ds(row0, _RT_EDGE)], ew_v)

    @pl.loop(0, _RT_EDGE)
    def _scat(j):
        pltpu.sync_copy(ew_v.at[j], deg_sh.at[dst_v.at[j]], add=True)

    plsc.subcore_barrier()

    @pl.when(s == 0)
    def _write():
        pltpu.sync_copy(deg_sh, out_hbm.at[c])


_sc_deg = pl.kernel(
    _sc_deg_body,
    out_type=jax.ShapeDtypeStruct((2, _DEG_PAD), _F32),
    mesh=_SC_MESH,
    scratch_types=[
        pltpu.VMEM((_RT_EDGE, 128), jnp.int32),
        pltpu.VMEM((_RT_EDGE, 128), _F32),
        pltpu.VMEM((640,), _F32),
        pltpu.VMEM_SHARED((_DEG_PAD,), _F32),
    ],
)


def _sc_spmm_body(mode_cols, table_hbm, src_hbm, dst_hbm, ew_hbm, out_hbm,
                  src_v, dst_v, ew_v, rows_v, agg_sh, sem):
    c = lax.axis_index("c")
    s = lax.axis_index("s")
    rows_t = _RT_COLS if mode_cols else _RT_EDGE

    @pl.loop(0, 128)
    def _zero(k):
        for q in range(8):
            rows_v[k, pl.ds(q * 16, 16)] = jnp.zeros((16,), _F32)

    for t in range(5):
        pltpu.sync_copy(rows_v,
                        agg_sh.at[pl.ds(s * _NPT + t * 128, 128)])
    plsc.subcore_barrier()

    stg = rows_t // 5
    if mode_cols:
        row0 = s * _RT_COLS
    else:
        row0 = c * (_EROWS // 2) + s * _RT_EDGE

    @pl.loop(0, 5)
    def _blk(b):
        r0 = pl.multiple_of(row0 + b * stg, 8)
        if mode_cols:
            pltpu.sync_copy(src_hbm.at[c, pl.ds(r0, stg)], src_v)
        else:
            pltpu.sync_copy(src_hbm.at[0, pl.ds(r0, stg)], src_v)
        pltpu.sync_copy(dst_hbm.at[pl.ds(r0, stg)], dst_v)
        pltpu.sync_copy(ew_hbm.at[pl.ds(r0, stg)], ew_v)

        @pl.loop(0, stg)
        def _edge_chunk(j):
            pltpu.async_copy(table_hbm.at[src_v.at[j]], rows_v, sem).wait()

            @pl.loop(0, 8)
            def _scale(t):
                wv = ew_v[j, pl.ds(t * 16, 16)]
                for k in range(16):
                    w = wv[k]
                    r = t * 16 + k
                    for q in range(8):
                        rows_v[r, pl.ds(q * 16, 16)] = (
                            rows_v[r, pl.ds(q * 16, 16)] * w)

            pltpu.sync_copy(rows_v, agg_sh.at[dst_v.at[j]], add=True)

    plsc.subcore_barrier()
    pltpu.sync_copy(agg_sh.at[pl.ds(s * _NPT, _NPT)],
                    out_hbm.at[c, pl.ds(s * _NPT, _NPT)])


def _make_spmm(mode_cols):
    rows_t = _RT_COLS if mode_cols else _RT_EDGE
    return pl.kernel(
        functools.partial(_sc_spmm_body, mode_cols),
        out_type=jax.ShapeDtypeStruct((2, _NP, 128), _F32),
        mesh=_SC_MESH,
        scratch_types=[
            pltpu.VMEM((rows_t // 5, 128), jnp.int32),
            pltpu.VMEM((rows_t // 5, 128), jnp.int32),
            pltpu.VMEM((rows_t // 5, 128), _F32),
            pltpu.VMEM((128, 128), _F32),
            pltpu.VMEM_SHARED((_NP, 128), _F32),
            pltpu.SemaphoreType.DMA,
        ],
    )


_sc_spmm_cols = _make_spmm(True)
_sc_spmm_edges = _make_spmm(False)


def _dinv_from_degT(degT_ref):
    deg = degT_ref[:, 0:1] + degT_ref[:, 1:2]
    safe = jnp.where(deg > 0.0, deg, 1.0)
    return jnp.where(deg > 0.0, lax.rsqrt(safe), 0.0)


def _g1_body(x_ref, w_ref, degT_ref, out_ref):
    dinv = _dinv_from_degT(degT_ref)
    h = jnp.dot(x_ref[...], w_ref[...], preferred_element_type=_F32)
    out_ref[...] = h * dinv


def _mid_body(s1a_ref, s1b_ref, w2a_ref, w2b_ref, b1r_ref, degT_ref, out_ref):
    dinv = _dinv_from_degT(degT_ref)
    z1a = jnp.maximum(s1a_ref[0] * dinv + b1r_ref[0:1, :], 0.0)
    z1b = jnp.maximum(s1b_ref[0] * dinv + b1r_ref[1:2, :], 0.0)
    h2 = (jnp.dot(z1a, w2a_ref[...], preferred_element_type=_F32)
          + jnp.dot(z1b, w2b_ref[...], preferred_element_type=_F32))
    out_ref[...] = h2 * dinv


def _z2_body(s2a_ref, s2b_ref, b2_ref, degT_ref, out_ref):
    dinv = _dinv_from_degT(degT_ref)
    out_ref[...] = (s2a_ref[0] + s2b_ref[0]) * dinv + b2_ref[...]


def _head_body(lat_ref, out_ref):
    for t in range(_HB):
        a = lat_ref[t]
        g = lax.dot_general(a, a, (((1,), (1,)), ((), ())),
                            preferred_element_type=_F32)
        out_ref[t] = 1.0 / (1.0 + jnp.exp(-g))


def _g1cat(x, W1, degT):
    return pl.pallas_call(
        _g1_body,
        grid=(_NRB, 2),
        in_specs=[
            pl.BlockSpec((_RB, _DIN), lambda i, j: (i, 0)),
            pl.BlockSpec((_DIN, 128), lambda i, j: (0, j)),
            pl.BlockSpec((_RB, 2), lambda i, j: (i, 0)),
        ],
        out_specs=pl.BlockSpec((_RB, 128), lambda i, j: (j * _NRB + i, 0)),
        out_shape=jax.ShapeDtypeStruct((2 * _N, 128), _F32),
    )(x, W1, degT)


def _g2(s1, W2, b1r, degT):
    return pl.pallas_call(
        _mid_body,
        grid=(_NRB,),
        in_specs=[
            pl.BlockSpec((1, _RB, 128), lambda i: (0, i, 0)),
            pl.BlockSpec((1, _RB, 128), lambda i: (1, i, 0)),
            pl.BlockSpec((128, 128), lambda i: (0, 0)),
            pl.BlockSpec((128, 128), lambda i: (1, 0)),
            pl.BlockSpec((2, 128), lambda i: (0, 0)),
            pl.BlockSpec((_RB, 2), lambda i: (i, 0)),
        ],
        out_specs=pl.BlockSpec((_RB, 128), lambda i: (i, 0)),
        out_shape=jax.ShapeDtypeStruct((_N, 128), _F32),
    )(s1, s1, W2, W2, b1r, degT)


def _lat(s2, b2r, degT):
    return pl.pallas_call(
        _z2_body,
        grid=(_NRB,),
        in_specs=[
            pl.BlockSpec((1, _RB, 128), lambda i: (0, i, 0)),
            pl.BlockSpec((1, _RB, 128), lambda i: (1, i, 0)),
            pl.BlockSpec((1, 128), lambda i: (0, 0)),
            pl.BlockSpec((_RB, 2), lambda i: (i, 0)),
        ],
        out_specs=pl.BlockSpec((_RB, 128), lambda i: (i, 0)),
        out_shape=jax.ShapeDtypeStruct((_N, 128), _F32),
    )(s2, s2, b2r, degT)


def _head(latB):
    return pl.pallas_call(
        _head_body,
        grid=(_M // _HB,),
        in_specs=[pl.BlockSpec((_HB, _M, _DOUT), lambda i: (i, 0, 0))],
        out_specs=pl.BlockSpec((_HB, _M, _M), lambda i: (i, 0, 0)),
        out_shape=jax.ShapeDtypeStruct((_M, _M, _M), _F32),
    )(latB)


def kernel(x, edge_index, edge_attr, W1, b1, W2, b2):
    src = edge_index[0].astype(jnp.int32)
    dst = edge_index[1].astype(jnp.int32)
    ew = edge_attr

    pad = _EP - _E
    srcp = jnp.concatenate([src, jnp.zeros((pad,), jnp.int32)])
    dstp = jnp.concatenate([dst, jnp.zeros((pad,), jnp.int32)])
    ewp = jnp.concatenate([ew, jnp.zeros((pad,), _F32)])
    src2x = jnp.stack([srcp, srcp + _N]).reshape(2, _EROWS, 128)
    dst2d = dstp.reshape(_EROWS, 128)
    ew2d = ewp.reshape(_EROWS, 128)

    degp = _sc_deg(dst2d, ew2d)
    degT = jnp.transpose(degp[:, :_N])

    g1cat = _g1cat(x, W1, degT)
    s1 = _sc_spmm_cols(g1cat, src2x, dst2d, ew2d)[:, :_N]

    g2 = _g2(s1, W2, b1.reshape(2, 128), degT)
    s2 = _sc_spmm_edges(g2, src2x, dst2d, ew2d)[:, :_N]

    lat = _lat(s2, b2.reshape(1, 128), degT)
    return _head(lat.reshape(_M, _M, _DOUT))


# speedup vs baseline: 6.4601x; 1.1682x over previous
import functools

import jax
import jax.numpy as jnp
from jax import lax
from jax.experimental import pallas as pl
from jax.experimental.pallas import tpu as pltpu
from jax.experimental.pallas import tpu_sc as plsc

_N = 10000
_DIN = 128
_DH = 256
_DOUT = 128
_M = 100
_RB = 1000
_NRB = _N // _RB
_HB = 10

_F32 = jnp.float32

_E = 320000
_EP = 327680
_EROWS = _EP // 128
_RT_COLS = _EROWS // 16
_RT_EDGE = _EROWS // 32
_NP = 10240
_NPT = _NP // 16
_DEG_PAD = _NP

_SC_MESH = plsc.VectorSubcoreMesh(core_axis_name="c", subcore_axis_name="s")


def _sc_deg_body(dst_hbm, ew_hbm, out_hbm, dst_v, ew_v, zbuf, deg_sh):
    c = lax.axis_index("c")
    s = lax.axis_index("s")

    @pl.loop(0, 40)
    def _zero(i):
        zbuf[pl.ds(i * 16, 16)] = jnp.zeros((16,), _F32)

    pltpu.sync_copy(zbuf, deg_sh.at[pl.ds(s * 640, 640)])
    plsc.subcore_barrier()

    row0 = c * (_EROWS // 2) + s * _RT_EDGE
    pltpu.sync_copy(dst_hbm.at[pl.ds(row0, _RT_EDGE)], dst_v)
    pltpu.sync_copy(ew_hbm.at[pl.ds(row0, _RT_EDGE)], ew_v)

    @pl.loop(0, _RT_EDGE)
    def _scat(j):
        pltpu.sync_copy(ew_v.at[j], deg_sh.at[dst_v.at[j]], add=True)

    plsc.subcore_barrier()

    @pl.when(s == 0)
    def _write():
        pltpu.sync_copy(deg_sh, out_hbm.at[c])


_sc_deg = pl.kernel(
    _sc_deg_body,
    out_type=jax.ShapeDtypeStruct((2, _DEG_PAD), _F32),
    mesh=_SC_MESH,
    scratch_types=[
        pltpu.VMEM((_RT_EDGE, 128), jnp.int32),
        pltpu.VMEM((_RT_EDGE, 128), _F32),
        pltpu.VMEM((640,), _F32),
        pltpu.VMEM_SHARED((_DEG_PAD,), _F32),
    ],
)


def _sc_spmm_body(mode_cols, table_hbm, src_hbm, dst_hbm, ew_hbm, out_hbm,
                  src_v, dst_v, ew_v, rows0, rows1,
                  agg_sh, semg0, semg1, sems0, sems1):
    c = lax.axis_index("c")
    s = lax.axis_index("s")
    rows_t = _RT_COLS if mode_cols else _RT_EDGE

    @pl.loop(0, 128)
    def _zero(k):
        for q in range(8):
            rows0[k, pl.ds(q * 16, 16)] = jnp.zeros((16,), _F32)

    for t in range(5):
        pltpu.sync_copy(rows0,
                        agg_sh.at[pl.ds(s * _NPT + t * 128, 128)])
    plsc.subcore_barrier()

    stg = rows_t // 5
    npair = stg // 2
    if mode_cols:
        row0 = s * _RT_COLS
    else:
        row0 = c * (_EROWS // 2) + s * _RT_EDGE

    def _scale(buf, j):
        @pl.loop(0, 8)
        def _scale_grp(t):
            wv = ew_v[j, pl.ds(t * 16, 16)]
            for k in range(16):
                w = wv[k]
                r = t * 16 + k
                for q in range(8):
                    buf[r, pl.ds(q * 16, 16)] = buf[r, pl.ds(q * 16, 16)] * w

    @pl.loop(0, 5)
    def _blk(b):
        r0 = pl.multiple_of(row0 + b * stg, 8)
        if mode_cols:
            pltpu.sync_copy(src_hbm.at[c, pl.ds(r0, stg)], src_v)
        else:
            pltpu.sync_copy(src_hbm.at[0, pl.ds(r0, stg)], src_v)
        pltpu.sync_copy(dst_hbm.at[pl.ds(r0, stg)], dst_v)
        pltpu.sync_copy(ew_hbm.at[pl.ds(r0, stg)], ew_v)

        pltpu.async_copy(table_hbm.at[src_v.at[0]], rows0, semg0)

        @pl.loop(0, npair)
        def _pair(jj):
            j0 = jj * 2
            j1 = j0 + 1

            @pl.when(jj > 0)
            def _w1():
                pltpu.make_async_copy(
                    rows1, agg_sh.at[dst_v.at[0]], sems1).wait()

            pltpu.async_copy(table_hbm.at[src_v.at[j1]], rows1, semg1)

            pltpu.make_async_copy(
                table_hbm.at[src_v.at[j0]], rows0, semg0).wait()
            _scale(rows0, j0)
            pltpu.async_copy(rows0, agg_sh.at[dst_v.at[j0]], sems0, add=True)

            pltpu.make_async_copy(
                table_hbm.at[src_v.at[j1]], rows1, semg1).wait()
            _scale(rows1, j1)
            pltpu.async_copy(rows1, agg_sh.at[dst_v.at[j1]], sems1, add=True)

            @pl.when(jj < npair - 1)
            def _next():
                pltpu.make_async_copy(
                    rows0, agg_sh.at[dst_v.at[0]], sems0).wait()
                pltpu.async_copy(table_hbm.at[src_v.at[j0 + 2]], rows0, semg0)

        pltpu.make_async_copy(rows0, agg_sh.at[dst_v.at[0]], sems0).wait()
        pltpu.make_async_copy(rows1, agg_sh.at[dst_v.at[0]], sems1).wait()

    plsc.subcore_barrier()
    pltpu.sync_copy(agg_sh.at[pl.ds(s * _NPT, _NPT)],
                    out_hbm.at[c, pl.ds(s * _NPT, _NPT)])


def _make_spmm(mode_cols):
    rows_t = _RT_COLS if mode_cols else _RT_EDGE
    return pl.kernel(
        functools.partial(_sc_spmm_body, mode_cols),
        out_type=jax.ShapeDtypeStruct((2, _NP, 128), _F32),
        mesh=_SC_MESH,
        scratch_types=[
            pltpu.VMEM((rows_t // 5, 128), jnp.int32),
            pltpu.VMEM((rows_t // 5, 128), jnp.int32),
            pltpu.VMEM((rows_t // 5, 128), _F32),
            pltpu.VMEM((128, 128), _F32),
            pltpu.VMEM((128, 128), _F32),
            pltpu.VMEM_SHARED((_NP, 128), _F32),
            pltpu.SemaphoreType.DMA,
            pltpu.SemaphoreType.DMA,
            pltpu.SemaphoreType.DMA,
            pltpu.SemaphoreType.DMA,
        ],
    )


_sc_spmm_cols = _make_spmm(True)
_sc_spmm_edges = _make_spmm(False)


def _dinv_from_degT(degT_ref):
    deg = degT_ref[:, 0:1] + degT_ref[:, 1:2]
    safe = jnp.where(deg > 0.0, deg, 1.0)
    return jnp.where(deg > 0.0, lax.rsqrt(safe), 0.0)


def _g1_body(x_ref, w_ref, degT_ref, out_ref):
    dinv = _dinv_from_degT(degT_ref)
    h = jnp.dot(x_ref[...], w_ref[...], preferred_element_type=_F32)
    out_ref[...] = h * dinv


def _mid_body(s1a_ref, s1b_ref, w2a_ref, w2b_ref, b1r_ref, degT_ref, out_ref):
    dinv = _dinv_from_degT(degT_ref)
    z1a = jnp.maximum(s1a_ref[0] * dinv + b1r_ref[0:1, :], 0.0)
    z1b = jnp.maximum(s1b_ref[0] * dinv + b1r_ref[1:2, :], 0.0)
    h2 = (jnp.dot(z1a, w2a_ref[...], preferred_element_type=_F32)
          + jnp.dot(z1b, w2b_ref[...], preferred_element_type=_F32))
    out_ref[...] = h2 * dinv


def _z2_body(s2a_ref, s2b_ref, b2_ref, degT_ref, out_ref):
    dinv = _dinv_from_degT(degT_ref)
    out_ref[...] = (s2a_ref[0] + s2b_ref[0]) * dinv + b2_ref[...]


def _head_body(lat_ref, out_ref):
    for t in range(_HB):
        a = lat_ref[t]
        g = lax.dot_general(a, a, (((1,), (1,)), ((), ())),
                            preferred_element_type=_F32)
        out_ref[t] = 1.0 / (1.0 + jnp.exp(-g))


def _g1cat(x, W1, degT):
    return pl.pallas_call(
        _g1_body,
        grid=(_NRB, 2),
        in_specs=[
            pl.BlockSpec((_RB, _DIN), lambda i, j: (i, 0)),
            pl.BlockSpec((_DIN, 128), lambda i, j: (0, j)),
            pl.BlockSpec((_RB, 2), lambda i, j: (i, 0)),
        ],
        out_specs=pl.BlockSpec((_RB, 128), lambda i, j: (j * _NRB + i, 0)),
        out_shape=jax.ShapeDtypeStruct((2 * _N, 128), _F32),
    )(x, W1, degT)


def _g2(s1, W2, b1r, degT):
    return pl.pallas_call(
        _mid_body,
        grid=(_NRB,),
        in_specs=[
            pl.BlockSpec((1, _RB, 128), lambda i: (0, i, 0)),
            pl.BlockSpec((1, _RB, 128), lambda i: (1, i, 0)),
            pl.BlockSpec((128, 128), lambda i: (0, 0)),
            pl.BlockSpec((128, 128), lambda i: (1, 0)),
            pl.BlockSpec((2, 128), lambda i: (0, 0)),
            pl.BlockSpec((_RB, 2), lambda i: (i, 0)),
        ],
        out_specs=pl.BlockSpec((_RB, 128), lambda i: (i, 0)),
        out_shape=jax.ShapeDtypeStruct((_N, 128), _F32),
    )(s1, s1, W2, W2, b1r, degT)


def _lat(s2, b2r, degT):
    return pl.pallas_call(
        _z2_body,
        grid=(_NRB,),
        in_specs=[
            pl.BlockSpec((1, _RB, 128), lambda i: (0, i, 0)),
            pl.BlockSpec((1, _RB, 128), lambda i: (1, i, 0)),
            pl.BlockSpec((1, 128), lambda i: (0, 0)),
            pl.BlockSpec((_RB, 2), lambda i: (i, 0)),
        ],
        out_specs=pl.BlockSpec((_RB, 128), lambda i: (i, 0)),
        out_shape=jax.ShapeDtypeStruct((_N, 128), _F32),
    )(s2, s2, b2r, degT)


def _head(latB):
    return pl.pallas_call(
        _head_body,
        grid=(_M // _HB,),
        in_specs=[pl.BlockSpec((_HB, _M, _DOUT), lambda i: (i, 0, 0))],
        out_specs=pl.BlockSpec((_HB, _M, _M), lambda i: (i, 0, 0)),
        out_shape=jax.ShapeDtypeStruct((_M, _M, _M), _F32),
    )(latB)


def kernel(x, edge_index, edge_attr, W1, b1, W2, b2):
    src = edge_index[0].astype(jnp.int32)
    dst = edge_index[1].astype(jnp.int32)
    ew = edge_attr

    pad = _EP - _E
    srcp = jnp.concatenate([src, jnp.zeros((pad,), jnp.int32)])
    dstp = jnp.concatenate([dst, jnp.zeros((pad,), jnp.int32)])
    ewp = jnp.concatenate([ew, jnp.zeros((pad,), _F32)])
    src2x = jnp.stack([srcp, srcp + _N]).reshape(2, _EROWS, 128)
    dst2d = dstp.reshape(_EROWS, 128)
    ew2d = ewp.reshape(_EROWS, 128)

    degp = _sc_deg(dst2d, ew2d)
    degT = jnp.transpose(degp[:, :_N])

    g1cat = _g1cat(x, W1, degT)
    s1 = _sc_spmm_cols(g1cat, src2x, dst2d, ew2d)[:, :_N]

    g2 = _g2(s1, W2, b1.reshape(2, 128), degT)
    s2 = _sc_spmm_edges(g2, src2x, dst2d, ew2d)[:, :_N]

    lat = _lat(s2, b2.reshape(1, 128), degT)
    return _head(lat.reshape(_M, _M, _DOUT))


# speedup vs baseline: 15.5268x; 2.4035x over previous
import functools

import jax
import jax.numpy as jnp
from jax import lax
from jax.experimental import pallas as pl
from jax.experimental.pallas import tpu as pltpu
from jax.experimental.pallas import tpu_sc as plsc

_N = 10000
_DIN = 128
_DH = 256
_DOUT = 128
_M = 100
_RB = 1000
_NRB = _N // _RB
_HB = 10

_F32 = jnp.float32

_E = 320000
_EP = 327680
_EROWS = _EP // 128
_RT_COLS = _EROWS // 16
_RT_EDGE = _EROWS // 32
_NP = 10240
_NPT = _NP // 16
_DEG_PAD = _NP

_SC_MESH = plsc.VectorSubcoreMesh(core_axis_name="c", subcore_axis_name="s")


def _sc_deg_body(dst_hbm, ew_hbm, out_hbm, dst_v, ew_v, zbuf, deg_sh):
    c = lax.axis_index("c")
    s = lax.axis_index("s")

    @pl.loop(0, 40)
    def _zero(i):
        zbuf[pl.ds(i * 16, 16)] = jnp.zeros((16,), _F32)

    pltpu.sync_copy(zbuf, deg_sh.at[pl.ds(s * 640, 640)])
    plsc.subcore_barrier()

    row0 = c * (_EROWS // 2) + s * _RT_EDGE
    pltpu.sync_copy(dst_hbm.at[pl.ds(row0, _RT_EDGE)], dst_v)
    pltpu.sync_copy(ew_hbm.at[pl.ds(row0, _RT_EDGE)], ew_v)

    @pl.loop(0, _RT_EDGE)
    def _scat(j):
        pltpu.sync_copy(ew_v.at[j], deg_sh.at[dst_v.at[j]], add=True)

    plsc.subcore_barrier()

    @pl.when(s == 0)
    def _write():
        pltpu.sync_copy(deg_sh, out_hbm.at[c])


_sc_deg = pl.kernel(
    _sc_deg_body,
    out_type=jax.ShapeDtypeStruct((2, _DEG_PAD), _F32),
    mesh=_SC_MESH,
    scratch_types=[
        pltpu.VMEM((_RT_EDGE, 128), jnp.int32),
        pltpu.VMEM((_RT_EDGE, 128), _F32),
        pltpu.VMEM((640,), _F32),
        pltpu.VMEM_SHARED((_DEG_PAD,), _F32),
    ],
)


def _sc_spmm_body(mode_cols, table_hbm, src_hbm, dst_hbm, ew_hbm, out_hbm,
                  src_v, dst_v, ew_v, rows0, rows1,
                  agg_sh, semg0, semg1, sems0, sems1):
    c = lax.axis_index("c")
    s = lax.axis_index("s")
    rows_t = _RT_COLS if mode_cols else _RT_EDGE

    @pl.loop(0, 128)
    def _zero(k):
        for q in range(8):
            rows0[k, pl.ds(q * 16, 16)] = jnp.zeros((16,), _F32)

    for t in range(5):
        pltpu.sync_copy(rows0,
                        agg_sh.at[pl.ds(s * _NPT + t * 128, 128)])
    plsc.subcore_barrier()

    stg = rows_t // 5
    npair = stg // 2
    if mode_cols:
        row0 = s * _RT_COLS
    else:
        row0 = c * (_EROWS // 2) + s * _RT_EDGE

    def _scale(buf, j):
        @pl.loop(0, 8)
        def _scale_grp(t):
            wv = ew_v[j, pl.ds(t * 16, 16)]
            for k in range(16):
                w = wv[k]
                r = t * 16 + k
                for q in range(8):
                    buf[r, pl.ds(q * 16, 16)] = buf[r, pl.ds(q * 16, 16)] * w

    @pl.loop(0, 5)
    def _blk(b):
        r0 = pl.multiple_of(row0 + b * stg, 8)
        if mode_cols:
            pltpu.sync_copy(src_hbm.at[c, pl.ds(r0, stg)], src_v)
        else:
            pltpu.sync_copy(src_hbm.at[0, pl.ds(r0, stg)], src_v)
        pltpu.sync_copy(dst_hbm.at[pl.ds(r0, stg)], dst_v)
        pltpu.sync_copy(ew_hbm.at[pl.ds(r0, stg)], ew_v)

        pltpu.async_copy(table_hbm.at[src_v.at[0]], rows0, semg0)

        @pl.loop(0, npair)
        def _pair(jj):
            j0 = jj * 2
            j1 = j0 + 1

            @pl.when(jj > 0)
            def _w1():
                pltpu.make_async_copy(
                    rows1, agg_sh.at[dst_v.at[0]], sems1).wait()

            pltpu.async_copy(table_hbm.at[src_v.at[j1]], rows1, semg1)

            pltpu.make_async_copy(
                table_hbm.at[src_v.at[j0]], rows0, semg0).wait()
            _scale(rows0, j0)
            pltpu.async_copy(rows0, agg_sh.at[dst_v.at[j0]], sems0, add=True)

            pltpu.make_async_copy(
                table_hbm.at[src_v.at[j1]], rows1, semg1).wait()
            _scale(rows1, j1)
            pltpu.async_copy(rows1, agg_sh.at[dst_v.at[j1]], sems1, add=True)

            @pl.when(jj < npair - 1)
            def _next():
                pltpu.make_async_copy(
                    rows0, agg_sh.at[dst_v.at[0]], sems0).wait()
                pltpu.async_copy(table_hbm.at[src_v.at[j0 + 2]], rows0, semg0)

        pltpu.make_async_copy(rows0, agg_sh.at[dst_v.at[0]], sems0).wait()
        pltpu.make_async_copy(rows1, agg_sh.at[dst_v.at[0]], sems1).wait()

    plsc.subcore_barrier()
    pltpu.sync_copy(agg_sh.at[pl.ds(s * _NPT, _NPT)],
                    out_hbm.at[c, pl.ds(s * _NPT, _NPT)])


def _make_spmm(mode_cols):
    rows_t = _RT_COLS if mode_cols else _RT_EDGE
    return pl.kernel(
        functools.partial(_sc_spmm_body, mode_cols),
        out_type=jax.ShapeDtypeStruct((2, _NP, 128), _F32),
        mesh=_SC_MESH,
        scratch_types=[
            pltpu.VMEM((rows_t // 5, 128), jnp.int32),
            pltpu.VMEM((rows_t // 5, 128), jnp.int32),
            pltpu.VMEM((rows_t // 5, 128), _F32),
            pltpu.VMEM((128, 128), _F32),
            pltpu.VMEM((128, 128), _F32),
            pltpu.VMEM_SHARED((_NP, 128), _F32),
            pltpu.SemaphoreType.DMA,
            pltpu.SemaphoreType.DMA,
            pltpu.SemaphoreType.DMA,
            pltpu.SemaphoreType.DMA,
        ],
    )


_sc_spmm_cols = _make_spmm(True)
_sc_spmm_edges = _make_spmm(False)


def _dinv_from_degT(degT_ref):
    deg = degT_ref[:, 0:1] + degT_ref[:, 1:2]
    safe = jnp.where(deg > 0.0, deg, 1.0)
    return jnp.where(deg > 0.0, lax.rsqrt(safe), 0.0)


def _g1_body(x_ref, w_ref, degT_ref, out_ref):
    dinv = _dinv_from_degT(degT_ref)
    h = jnp.dot(x_ref[...], w_ref[...], preferred_element_type=_F32)
    out_ref[...] = h * dinv


def _mid_body(s1a_ref, s1b_ref, w2a_ref, w2b_ref, b1r_ref, degT_ref, out_ref):
    dinv = _dinv_from_degT(degT_ref)
    z1a = jnp.maximum(s1a_ref[0] * dinv + b1r_ref[0:1, :], 0.0)
    z1b = jnp.maximum(s1b_ref[0] * dinv + b1r_ref[1:2, :], 0.0)
    h2 = (jnp.dot(z1a, w2a_ref[...], preferred_element_type=_F32)
          + jnp.dot(z1b, w2b_ref[...], preferred_element_type=_F32))
    out_ref[...] = h2 * dinv


def _z2_body(s2a_ref, s2b_ref, b2_ref, degT_ref, out_ref):
    dinv = _dinv_from_degT(degT_ref)
    out_ref[...] = (s2a_ref[0] + s2b_ref[0]) * dinv + b2_ref[...]


def _head_body(lat_ref, out_ref):
    for t in range(_HB):
        a = lat_ref[t]
        g = lax.dot_general(a, a, (((1,), (1,)), ((), ())),
                            preferred_element_type=_F32)
        out_ref[t] = 1.0 / (1.0 + jnp.exp(-g))


def _g1cat(x, W1, degT):
    return pl.pallas_call(
        _g1_body,
        grid=(_NRB, 2),
        in_specs=[
            pl.BlockSpec((_RB, _DIN), lambda i, j: (i, 0)),
            pl.BlockSpec((_DIN, 128), lambda i, j: (0, j)),
            pl.BlockSpec((_RB, 2), lambda i, j: (i, 0)),
        ],
        out_specs=pl.BlockSpec((_RB, 128), lambda i, j: (j * _NRB + i, 0)),
        out_shape=jax.ShapeDtypeStruct((2 * _N, 128), _F32),
    )(x, W1, degT)


def _g2(s1, W2, b1r, degT):
    return pl.pallas_call(
        _mid_body,
        grid=(_NRB,),
        in_specs=[
            pl.BlockSpec((1, _RB, 128), lambda i: (0, i, 0)),
            pl.BlockSpec((1, _RB, 128), lambda i: (1, i, 0)),
            pl.BlockSpec((128, 128), lambda i: (0, 0)),
            pl.BlockSpec((128, 128), lambda i: (1, 0)),
            pl.BlockSpec((2, 128), lambda i: (0, 0)),
            pl.BlockSpec((_RB, 2), lambda i: (i, 0)),
        ],
        out_specs=pl.BlockSpec((_RB, 128), lambda i: (i, 0)),
        out_shape=jax.ShapeDtypeStruct((_N, 128), _F32),
    )(s1, s1, W2, W2, b1r, degT)


def _lat(s2, b2r, degT):
    return pl.pallas_call(
        _z2_body,
        grid=(_NRB,),
        in_specs=[
            pl.BlockSpec((1, _RB, 128), lambda i: (0, i, 0)),
            pl.BlockSpec((1, _RB, 128), lambda i: (1, i, 0)),
            pl.BlockSpec((1, 128), lambda i: (0, 0)),
            pl.BlockSpec((_RB, 2), lambda i: (i, 0)),
        ],
        out_specs=pl.BlockSpec((_RB, 128), lambda i: (i, 0)),
        out_shape=jax.ShapeDtypeStruct((_N, 128), _F32),
    )(s2, s2, b2r, degT)


def _head(latB):
    return pl.pallas_call(
        _head_body,
        grid=(_M // _HB,),
        in_specs=[pl.BlockSpec((_HB, _M, _DOUT), lambda i: (i, 0, 0))],
        out_specs=pl.BlockSpec((_HB, _M, _M), lambda i: (i, 0, 0)),
        out_shape=jax.ShapeDtypeStruct((_M, _M, _M), _F32),
    )(latB)


def kernel(x, edge_index, edge_attr, W1, b1, W2, b2):
    src = edge_index[0].astype(jnp.int32)
    dst = edge_index[1].astype(jnp.int32)
    ew = edge_attr

    pad = _EP - _E
    pad_i = jnp.arange(pad, dtype=jnp.int32)
    srcp = jnp.concatenate([src, pad_i % _N])
    dstp = jnp.concatenate([dst, _N + pad_i % (_NP - _N)])
    ewp = jnp.concatenate([ew, jnp.zeros((pad,), _F32)])
    src2x = jnp.stack([srcp, srcp + _N]).reshape(2, _EROWS, 128)
    dst2d = dstp.reshape(_EROWS, 128)
    ew2d = ewp.reshape(_EROWS, 128)

    degp = _sc_deg(dst2d, ew2d)
    degT = jnp.transpose(degp[:, :_N])

    g1cat = _g1cat(x, W1, degT)
    s1 = _sc_spmm_cols(g1cat, src2x, dst2d, ew2d)[:, :_N]

    g2 = _g2(s1, W2, b1.reshape(2, 128), degT)
    s2 = _sc_spmm_edges(g2, src2x, dst2d, ew2d)[:, :_N]

    lat = _lat(s2, b2.reshape(1, 128), degT)
    return _head(lat.reshape(_M, _M, _DOUT))
